# baseline (speedup 1.0000x reference)
import math

import jax
import jax.numpy as jnp
from jax import lax
from jax.experimental import pallas as pl
from jax.experimental.pallas import tpu as pltpu
from jax.experimental.pallas import tpu_sc as plsc

D_MODEL = 512
SCALE = math.sqrt(D_MODEL)

NUM_CORES = 2
NUM_SUBCORES = 16
NUM_LANES = 16
NW = NUM_CORES * NUM_SUBCORES

GROUP = 2


def _sc_embedding(x, table):
    b, t = x.shape
    assert b % (NW * GROUP) == 0 and (GROUP * t) % 8 == 0 and GROUP * t <= 128
    b_per_w = b // NW
    n_groups = b_per_w // GROUP
    gsz = GROUP * t
    mesh = plsc.VectorSubcoreMesh(core_axis_name="c", subcore_axis_name="s")

    def body(idx_hbm, table_hbm, out_hbm,
             idx_v, raw0, raw1, stg0, stg1, gs0, gs1, st0, st1):
        wid = lax.axis_index("s") * NUM_CORES + lax.axis_index("c")
        pltpu.sync_copy(idx_hbm.at[wid], idx_v)
        base = wid * b_per_w
        last = n_groups - 1

        def fire_gather(g, raw, sem):
            pltpu.make_async_copy(
                table_hbm.at[idx_v.at[pl.ds(g * gsz, gsz)]], raw, sem
            ).start()

        def drain_gather(raw, sem):
            pltpu.make_async_copy(
                table_hbm.at[idx_v.at[pl.ds(0, gsz)]], raw, sem
            ).wait()

        def scale_into(raw, stg):
            @plsc.parallel_loop(0, t, unroll=2)
            def scale_row(r):
                for j in range(GROUP):
                    for c in range(D_MODEL // NUM_LANES):
                        sl = pl.ds(c * NUM_LANES, NUM_LANES)
                        stg[j, r, sl] = raw[j * t + r, sl] * SCALE

        def fire_store(g, stg, sem):
            pltpu.make_async_copy(
                stg, out_hbm.at[pl.ds(base + g * GROUP, GROUP)], sem
            ).start()

        def drain_store(stg, sem):
            pltpu.make_async_copy(
                stg, out_hbm.at[pl.ds(base, GROUP)], sem
            ).wait()

        def slot(g, raw, stg, gsem, ssem, first):
            drain_gather(raw, gsem)
            if not first:
                drain_store(stg, ssem)
            scale_into(raw, stg)
            fire_store(g, stg, ssem)
            fire_gather(jnp.minimum(g + 2, last), raw, gsem)

        fire_gather(0, raw0, gs0)
        fire_gather(1, raw1, gs1)

        slot(0, raw0, stg0, gs0, st0, True)
        slot(1, raw1, stg1, gs1, st1, True)

        def ring(i, _):
            slot(2 * i, raw0, stg0, gs0, st0, False)
            slot(2 * i + 1, raw1, stg1, gs1, st1, False)
            return 0

        lax.fori_loop(1, n_groups // 2, ring, 0)
        drain_store(stg0, st0)
        drain_store(stg1, st1)
        drain_gather(raw0, gs0)
        drain_gather(raw1, gs1)

    run = pl.kernel(
        body,
        out_type=jax.ShapeDtypeStruct((b, t, D_MODEL), jnp.float32),
        mesh=mesh,
        scratch_types=[
            pltpu.VMEM((b_per_w * t,), jnp.int32),
            pltpu.VMEM((gsz, D_MODEL), jnp.float32),
            pltpu.VMEM((gsz, D_MODEL), jnp.float32),
            pltpu.VMEM((GROUP, t, D_MODEL), jnp.float32),
            pltpu.VMEM((GROUP, t, D_MODEL), jnp.float32),
            pltpu.SemaphoreType.DMA,
            pltpu.SemaphoreType.DMA,
            pltpu.SemaphoreType.DMA,
            pltpu.SemaphoreType.DMA,
        ],
    )
    idx2 = x.astype(jnp.int32).reshape(NW, b_per_w * t)
    return run(idx2, table)


def kernel(x, table):
    return _sc_embedding(x, table)


# speedup vs baseline: 1.1467x; 1.1467x over previous
import math

import jax
import jax.numpy as jnp
from jax import lax
from jax.experimental import pallas as pl
from jax.experimental.pallas import tpu as pltpu
from jax.experimental.pallas import tpu_sc as plsc

D_MODEL = 512
SCALE = math.sqrt(D_MODEL)

NUM_CORES = 2
NUM_SUBCORES = 16
NUM_LANES = 16
NW = NUM_CORES * NUM_SUBCORES

GROUP = 2


def _sc_embedding(x, table):
    b, t = x.shape
    assert b % (NW * GROUP) == 0 and (GROUP * t) % 8 == 0 and GROUP * t <= 128
    b_per_w = b // NW
    n_groups = b_per_w // GROUP
    gsz = GROUP * t
    mesh = plsc.VectorSubcoreMesh(core_axis_name="c", subcore_axis_name="s")

    def body(idx_hbm, table_hbm, out_hbm,
             idx_v, raw0, raw1, stg0, stg1, gs0, gs1, st0, st1):
        wid = lax.axis_index("s") * NUM_CORES + lax.axis_index("c")
        pltpu.sync_copy(idx_hbm.at[wid], idx_v)
        base = wid * b_per_w
        last = n_groups - 1

        def fire_gather(g, raw, sem):
            pltpu.make_async_copy(
                table_hbm.at[idx_v.at[pl.ds(g * gsz, gsz)]], raw, sem
            ).start()

        def drain_gather(raw, sem):
            pltpu.make_async_copy(
                table_hbm.at[idx_v.at[pl.ds(0, gsz)]], raw, sem
            ).wait()

        def scale_into(raw, stg):
            @plsc.parallel_loop(0, t)
            def scale_row(r):
                for j in range(GROUP):
                    for c in range(D_MODEL // NUM_LANES):
                        sl = pl.ds(c * NUM_LANES, NUM_LANES)
                        stg[j, r, sl] = raw[j * t + r, sl] * SCALE

        def fire_store(g, stg, sem):
            pltpu.make_async_copy(
                stg, out_hbm.at[pl.ds(base + g * GROUP, GROUP)], sem
            ).start()

        def drain_store(stg, sem):
            pltpu.make_async_copy(
                stg, out_hbm.at[pl.ds(base, GROUP)], sem
            ).wait()

        def slot(g, raw, stg, gsem, ssem, first):
            drain_gather(raw, gsem)
            if not first:
                drain_store(stg, ssem)
            scale_into(raw, stg)
            fire_store(g, stg, ssem)
            fire_gather(jnp.minimum(g + 2, last), raw, gsem)

        fire_gather(0, raw0, gs0)
        fire_gather(1, raw1, gs1)

        slot(0, raw0, stg0, gs0, st0, True)
        slot(1, raw1, stg1, gs1, st1, True)

        def ring(i, _):
            slot(2 * i, raw0, stg0, gs0, st0, False)
            slot(2 * i + 1, raw1, stg1, gs1, st1, False)
            return 0

        lax.fori_loop(1, n_groups // 2, ring, 0)
        drain_store(stg0, st0)
        drain_store(stg1, st1)
        drain_gather(raw0, gs0)
        drain_gather(raw1, gs1)

    run = pl.kernel(
        body,
        out_type=jax.ShapeDtypeStruct((b, t, D_MODEL), jnp.float32),
        mesh=mesh,
        scratch_types=[
            pltpu.VMEM((b_per_w * t,), jnp.int32),
            pltpu.VMEM((gsz, D_MODEL), jnp.float32),
            pltpu.VMEM((gsz, D_MODEL), jnp.float32),
            pltpu.VMEM((GROUP, t, D_MODEL), jnp.float32),
            pltpu.VMEM((GROUP, t, D_MODEL), jnp.float32),
            pltpu.SemaphoreType.DMA,
            pltpu.SemaphoreType.DMA,
            pltpu.SemaphoreType.DMA,
            pltpu.SemaphoreType.DMA,
        ],
    )
    idx2 = x.astype(jnp.int32).reshape(NW, b_per_w * t)
    return run(idx2, table)


def kernel(x, table):
    return _sc_embedding(x, table)
